# baseline (speedup 1.0000x reference)
import functools

import jax
import jax.numpy as jnp
from jax import lax
from jax.experimental import pallas as pl
from jax.experimental.pallas import tpu as pltpu
from jax.experimental.pallas import tpu_sc as plsc


def _sc_gather_coeffs(t, sqrt_ac, sqrt_omac):
    B = t.shape[0]
    mesh = plsc.VectorSubcoreMesh(core_axis_name="c", subcore_axis_name="s")

    @functools.partial(
        pl.kernel,
        mesh=mesh,
        out_type=[
            jax.ShapeDtypeStruct((B,), jnp.float32),
            jax.ShapeDtypeStruct((B,), jnp.float32),
        ],
        scratch_types=[
            pltpu.VMEM((B,), jnp.int32),
            pltpu.VMEM((B,), jnp.float32),
            pltpu.VMEM((B,), jnp.float32),
            pltpu.SemaphoreType.DMA,
        ],
    )
    def gather_kernel(t_hbm, ac_hbm, omac_hbm, s_hbm, sm_hbm, idx_v, s_v, sm_v, sem):
        cid = lax.axis_index("c")
        sid = lax.axis_index("s")

        @pl.when(jnp.logical_and(cid == 0, sid == 0))
        def _():
            pltpu.sync_copy(t_hbm, idx_v)
            pltpu.async_copy(ac_hbm.at[idx_v], s_v, sem).wait()
            pltpu.async_copy(omac_hbm.at[idx_v], sm_v, sem).wait()
            pltpu.sync_copy(s_v, s_hbm)
            pltpu.sync_copy(sm_v, sm_hbm)

    return gather_kernel(t, sqrt_ac, sqrt_omac)


def _tc_combine(x, n, s, sm, bs=4):
    B, C, H, W = x.shape

    def body(s_ref, sm_ref, x_ref, n_ref, o_ref):
        i = pl.program_id(0)
        for j in range(bs):
            b = i * bs + j
            o_ref[j] = s_ref[b] * x_ref[j] + sm_ref[b] * n_ref[j]

    return pl.pallas_call(
        body,
        grid=(B // bs,),
        in_specs=[
            pl.BlockSpec(memory_space=pltpu.SMEM),
            pl.BlockSpec(memory_space=pltpu.SMEM),
            pl.BlockSpec((bs, C, H, W), lambda i: (i, 0, 0, 0)),
            pl.BlockSpec((bs, C, H, W), lambda i: (i, 0, 0, 0)),
        ],
        out_specs=pl.BlockSpec((bs, C, H, W), lambda i: (i, 0, 0, 0)),
        out_shape=jax.ShapeDtypeStruct((B, C, H, W), jnp.float32),
    )(s, sm, x, n)


def kernel(x_start, t, noise, sqrt_alphas_cumprod, sqrt_one_minus_alphas_cumprod):
    s, sm = _sc_gather_coeffs(
        t.astype(jnp.int32), sqrt_alphas_cumprod, sqrt_one_minus_alphas_cumprod
    )
    return _tc_combine(x_start, noise, s, sm)


# speedup vs baseline: 3.5830x; 3.5830x over previous
import functools

import jax
import jax.numpy as jnp
from jax import lax
from jax.experimental import pallas as pl
from jax.experimental.pallas import tpu as pltpu
from jax.experimental.pallas import tpu_sc as plsc


def _sc_gather_coeffs(t, sqrt_ac, sqrt_omac):
    B = t.shape[0]
    mesh = plsc.VectorSubcoreMesh(core_axis_name="c", subcore_axis_name="s")

    @functools.partial(
        pl.kernel,
        mesh=mesh,
        out_type=[
            jax.ShapeDtypeStruct((B,), jnp.float32),
            jax.ShapeDtypeStruct((B,), jnp.float32),
        ],
        scratch_types=[
            pltpu.VMEM((B,), jnp.int32),
            pltpu.VMEM((B,), jnp.float32),
            pltpu.VMEM((B,), jnp.float32),
            pltpu.SemaphoreType.DMA,
        ],
    )
    def gather_kernel(t_hbm, ac_hbm, omac_hbm, s_hbm, sm_hbm, idx_v, s_v, sm_v, sem):
        cid = lax.axis_index("c")
        sid = lax.axis_index("s")

        @pl.when(jnp.logical_and(cid == 0, sid == 0))
        def _():
            pltpu.sync_copy(t_hbm, idx_v)
            pltpu.async_copy(ac_hbm.at[idx_v], s_v, sem).wait()
            pltpu.async_copy(omac_hbm.at[idx_v], sm_v, sem).wait()
            pltpu.sync_copy(s_v, s_hbm)
            pltpu.sync_copy(sm_v, sm_hbm)

    return gather_kernel(t, sqrt_ac, sqrt_omac)


def _tc_combine(xT, nT, s2, sm2, rb):
    Rtot, B = xT.shape

    def body(s_ref, sm_ref, x_ref, n_ref, o_ref):
        o_ref[...] = s_ref[...] * x_ref[...] + sm_ref[...] * n_ref[...]

    return pl.pallas_call(
        body,
        grid=(Rtot // rb,),
        in_specs=[
            pl.BlockSpec((1, B), lambda i: (0, 0)),
            pl.BlockSpec((1, B), lambda i: (0, 0)),
            pl.BlockSpec((rb, B), lambda i: (i, 0)),
            pl.BlockSpec((rb, B), lambda i: (i, 0)),
        ],
        out_specs=pl.BlockSpec((rb, B), lambda i: (i, 0)),
        out_shape=jax.ShapeDtypeStruct((Rtot, B), jnp.float32),
    )(s2, sm2, xT, nT)


def kernel(x_start, t, noise, sqrt_alphas_cumprod, sqrt_one_minus_alphas_cumprod):
    s, sm = _sc_gather_coeffs(
        t.astype(jnp.int32), sqrt_alphas_cumprod, sqrt_one_minus_alphas_cumprod
    )
    B = x_start.shape[0]
    xT = jnp.transpose(x_start, (1, 2, 3, 0)).reshape(-1, B)
    nT = jnp.transpose(noise, (1, 2, 3, 0)).reshape(-1, B)
    outT = _tc_combine(xT, nT, s.reshape(1, B), sm.reshape(1, B), rb=4704)
    out = outT.reshape(x_start.shape[1:] + (B,)).transpose(3, 0, 1, 2)
    return out


# speedup vs baseline: 3.6232x; 1.0112x over previous
import functools

import jax
import jax.numpy as jnp
from jax import lax
from jax.experimental import pallas as pl
from jax.experimental.pallas import tpu as pltpu
from jax.experimental.pallas import tpu_sc as plsc


def _sc_gather_coeffs(t, sqrt_ac, sqrt_omac):
    B = t.shape[0]
    mesh = plsc.VectorSubcoreMesh(core_axis_name="c", subcore_axis_name="s")

    @functools.partial(
        pl.kernel,
        mesh=mesh,
        out_type=[
            jax.ShapeDtypeStruct((B,), jnp.float32),
            jax.ShapeDtypeStruct((B,), jnp.float32),
        ],
        scratch_types=[
            pltpu.VMEM((B,), jnp.int32),
            pltpu.VMEM((B,), jnp.float32),
            pltpu.VMEM((B,), jnp.float32),
            pltpu.SemaphoreType.DMA,
        ],
    )
    def gather_kernel(t_hbm, ac_hbm, omac_hbm, s_hbm, sm_hbm, idx_v, s_v, sm_v, sem):
        cid = lax.axis_index("c")
        sid = lax.axis_index("s")

        @pl.when(jnp.logical_and(cid == 0, sid == 0))
        def _():
            pltpu.sync_copy(t_hbm, idx_v)
            pltpu.async_copy(ac_hbm.at[idx_v], s_v, sem).wait()
            pltpu.async_copy(omac_hbm.at[idx_v], sm_v, sem).wait()
            pltpu.sync_copy(s_v, s_hbm)
            pltpu.sync_copy(sm_v, sm_hbm)

    return gather_kernel(t, sqrt_ac, sqrt_omac)


def _tc_combine(xT, nT, s2, sm2, rb):
    Rtot, B = xT.shape

    def body(s_ref, sm_ref, x_ref, n_ref, o_ref):
        o_ref[...] = s_ref[...] * x_ref[...] + sm_ref[...] * n_ref[...]

    return pl.pallas_call(
        body,
        grid=(Rtot // rb,),
        in_specs=[
            pl.BlockSpec((1, B), lambda i: (0, 0)),
            pl.BlockSpec((1, B), lambda i: (0, 0)),
            pl.BlockSpec((rb, B), lambda i: (i, 0)),
            pl.BlockSpec((rb, B), lambda i: (i, 0)),
        ],
        out_specs=pl.BlockSpec((rb, B), lambda i: (i, 0)),
        out_shape=jax.ShapeDtypeStruct((Rtot, B), jnp.float32),
    )(s2, sm2, xT, nT)


def kernel(x_start, t, noise, sqrt_alphas_cumprod, sqrt_one_minus_alphas_cumprod):
    s, sm = _sc_gather_coeffs(
        t.astype(jnp.int32), sqrt_alphas_cumprod, sqrt_one_minus_alphas_cumprod
    )
    B = x_start.shape[0]
    xT = jnp.transpose(x_start, (1, 2, 3, 0)).reshape(-1, B)
    nT = jnp.transpose(noise, (1, 2, 3, 0)).reshape(-1, B)
    outT = _tc_combine(xT, nT, s.reshape(1, B), sm.reshape(1, B), rb=9408)
    out = outT.reshape(x_start.shape[1:] + (B,)).transpose(3, 0, 1, 2)
    return out
